# Initial kernel scaffold; baseline (speedup 1.0000x reference)
import jax
import jax.numpy as jnp
from jax.experimental import pallas as pl


def kernel(fields, tables, lin_w, r, bias):
    raise NotImplementedError("write your pallas kernel here")


# speedup vs baseline: 1.6990x; 1.6990x over previous
import functools

import jax
import jax.numpy as jnp
from jax import lax
from jax.experimental import pallas as pl
from jax.experimental.pallas import tpu as pltpu
from jax.experimental.pallas import tpu_sc as plsc

N_FIELDS = 26
VOCAB = 100000
EMB = 16
BATCH = 4096

_NW = 32
_CW = 128
_CHUNKS = (N_FIELDS * BATCH) // (_NW * _CW)


_CPAD = 32


def _sc_gather(tables_flat, idx):
    mesh = plsc.VectorSubcoreMesh(core_axis_name="c", subcore_axis_name="s")

    @functools.partial(
        pl.kernel,
        mesh=mesh,
        out_type=jax.ShapeDtypeStruct((_NW * _CHUNKS, _CW, EMB), jnp.float32),
        scratch_types=[
            pltpu.VMEM((_CPAD, _CW), jnp.int32),
            pltpu.VMEM((_CHUNKS, _CW, EMB), jnp.float32),
            pltpu.SemaphoreType.DMA,
        ],
        compiler_params=pltpu.CompilerParams(use_tc_tiling_on_sc=False),
    )
    def gather_kernel(tab_hbm, idx_hbm, out_hbm, idx_v, rows_v, sem):
        wid = lax.axis_index("s") * 2 + lax.axis_index("c")
        pltpu.sync_copy(idx_hbm.at[pl.ds(wid * _CPAD, _CPAD)], idx_v)
        copies = [
            pltpu.async_copy(tab_hbm.at[idx_v.at[j]], rows_v.at[j], sem)
            for j in range(_CHUNKS)
        ]
        for cp in copies:
            cp.wait()
        pltpu.sync_copy(rows_v, out_hbm.at[pl.ds(wid * _CHUNKS, _CHUNKS)])

    return gather_kernel(tables_flat, idx)


def _interact_body(v_ref, m_ref, w_ref, b_ref, o_ref):
    v = v_ref[...]
    a = jnp.dot(v, m_ref[...], preferred_element_type=jnp.float32)
    a = a + w_ref[...]
    o_ref[...] = jnp.sum(v * a, axis=1, keepdims=True) + b_ref[0]


def _interact(v, m, lin_w, bias):
    blk = 512
    return pl.pallas_call(
        _interact_body,
        grid=(BATCH // blk,),
        in_specs=[
            pl.BlockSpec((blk, N_FIELDS * EMB), lambda i: (i, 0)),
            pl.BlockSpec((N_FIELDS * EMB, N_FIELDS * EMB), lambda i: (0, 0)),
            pl.BlockSpec((1, N_FIELDS * EMB), lambda i: (0, 0)),
            pl.BlockSpec(memory_space=pltpu.SMEM),
        ],
        out_specs=pl.BlockSpec((blk, 1), lambda i: (i, 0)),
        out_shape=jax.ShapeDtypeStruct((BATCH, 1), jnp.float32),
    )(v, m, lin_w, bias)


def kernel(fields, tables, lin_w, r, bias):
    f2 = fields.reshape(N_FIELDS, BATCH).astype(jnp.int32)
    offs = (jnp.arange(N_FIELDS, dtype=jnp.int32) * VOCAB)[:, None]
    idx = (f2 + offs).T.reshape(_NW, _CHUNKS, _CW)
    idx = jnp.pad(idx, ((0, 0), (0, _CPAD - _CHUNKS), (0, 0)))
    idx = idx.reshape(_NW * _CPAD, _CW)
    tables_flat = tables.reshape(N_FIELDS * VOCAB, EMB)
    rows = _sc_gather(tables_flat, idx)
    v = rows.reshape(BATCH, N_FIELDS * EMB)
    m = jnp.kron(jnp.triu(r, 1), jnp.eye(EMB, dtype=r.dtype))
    out = _interact(v, m, lin_w, bias.astype(jnp.float32))
    return out.reshape(BATCH)
